# baseline (speedup 1.0000x reference)
import jax
import jax.numpy as jnp
from jax import lax
from jax.experimental import pallas as pl
from jax.experimental.pallas import tpu as pltpu
from jax.experimental.pallas import tpu_sc as plsc

B, V, D, L = 1024, 1000, 32, 200
NW = 32
NBW = B // NW
VB = 20
NB = V // VB
NPAIR = NB // 2
LB = 4
NCH = L // LB
IDX_W = 128


def _body(w5, i2, o5, tsc, xb0, xb1, yb0, yb1, wx0, wx1, ib, gid, gb, yo,
          semg, semx0, semx1, semw0, semw1, semo):
    c = lax.axis_index("c")
    s = lax.axis_index("s")
    wid = s * 2 + c
    bg = wid >> 2
    bc0 = (wid & 3) * 32
    bfirst = bg * 128 + bc0
    lanes = lax.iota(jnp.int32, 16)
    zero = lanes * 0

    i_dg0 = lanes >> 3
    i_dr = lanes & 7
    bl0 = (bfirst + lanes) * V
    bl1 = (bfirst + 16 + lanes) * V

    def load_slab(v0, xb, semx):
        return pltpu.make_async_copy(
            w5.at[pl.ds(v0, VB), :, bg, :, pl.ds(bc0, 32)],
            xb.at[:, :, :, pl.ds(0, 32)], semx)

    def transpose_batch(v0, xb, yb, wx):
        @plsc.parallel_loop(0, VB, unroll=4)
        def pv(vp):
            xv = xb.at[vp]
            for b4 in range(0, 32, 4):
                gs = []
                for b in range(b4, b4 + 4):
                    ibv = zero + b
                    for h in range(2):
                        gs.append(plsc.load_gather(
                            xv, [i_dg0 + 2 * h, i_dr, ibv]))
                k = 0
                for b in range(b4, b4 + 4):
                    for h in range(2):
                        yb[vp * 32 + b, pl.ds(h * 16, 16)] = gs[k]
                        k += 1
            k0 = vp * 32
            wx[k0 >> 7, pl.ds(k0 & 127, 16)] = bl0 + (v0 + vp)
            k1 = k0 + 16
            wx[k1 >> 7, pl.ds(k1 & 127, 16)] = bl1 + (v0 + vp)

    def scatter_ops(yb, wx, semw):
        return [pltpu.make_async_copy(
            yb.at[pl.ds(p * 128, 128)], tsc.at[wx.at[p]], semw)
            for p in range(5)]

    load_slab(0, xb0, semx0).start()

    def pair(j, carry):
        v0a = j * (2 * VB)
        load_slab(v0a + VB, xb1, semx1).start()
        load_slab(v0a, xb0, semx0).wait()

        @pl.when(j > 0)
        def _():
            for op in scatter_ops(yb0, wx0, semw0):
                op.wait()

        transpose_batch(v0a, xb0, yb0, wx0)
        for op in scatter_ops(yb0, wx0, semw0):
            op.start()

        @pl.when(j + 1 < NPAIR)
        def _():
            load_slab(v0a + 2 * VB, xb0, semx0).start()

        load_slab(v0a + VB, xb1, semx1).wait()

        @pl.when(j > 0)
        def _():
            for op in scatter_ops(yb1, wx1, semw1):
                op.wait()

        transpose_batch(v0a + VB, xb1, yb1, wx1)
        for op in scatter_ops(yb1, wx1, semw1):
            op.start()
        return carry

    lax.fori_loop(0, NPAIR, pair, 0)
    for op in scatter_ops(yb0, wx0, semw0):
        op.wait()
    for op in scatter_ops(yb1, wx1, semw1):
        op.wait()

    pltpu.sync_copy(i2.at[:, pl.ds(bfirst, 32)], ib)

    def p2(i, carry):
        li = i >> 1
        h = i & 1
        base = (bfirst + h * 16 + lanes) * V
        vv = ib[li, pl.ds(h * 16, 16)]
        pos = li * 32 + h * 16
        gid[pos >> 7, pl.ds(pos & 127, 16)] = vv + base
        return carry

    lax.fori_loop(0, 2 * L, p2, 0)

    def fire_gather(ch, half):
        pltpu.make_async_copy(
            tsc.at[gid.at[ch]], gb.at[pl.ds(half * 128, 128)], semg).start()

    def wait_gather(half):
        pltpu.make_async_copy(
            tsc.at[gid.at[0]], gb.at[pl.ds(half * 128, 128)], semg).wait()

    def wb_op(ch):
        return pltpu.make_async_copy(
            yo.at[:, :, :, pl.ds(0, 32)],
            o5.at[pl.ds(ch * LB, LB), :, bg, :, pl.ds(bc0, 32)], semo)

    fire_gather(0, 0)

    def p23(ch, carry):
        half = ch & 1

        @pl.when(ch + 1 < NCH)
        def _():
            fire_gather(ch + 1, 1 - half)

        wait_gather(half)

        @pl.when(ch >= 1)
        def _():
            wb_op(ch - 1).wait()

        @plsc.parallel_loop(0, LB, unroll=4)
        def p3(lp):
            ilp = zero + lp
            for b4 in range(0, 32, 4):
                gs = []
                for b in range(b4, b4 + 4):
                    for h in range(2):
                        gs.append(gb[half * 128 + lp * 32 + b,
                                     pl.ds(h * 16, 16)])
                k = 0
                for b in range(b4, b4 + 4):
                    ibv = zero + b
                    for h in range(2):
                        plsc.store_scatter(
                            yo, [ilp, i_dg0 + 2 * h, i_dr, ibv], gs[k])
                        k += 1

        wb_op(ch).start()
        return carry

    lax.fori_loop(0, NCH, p23, 0)
    wb_op(NCH - 1).wait()


@jax.jit
def _run(w5, i2):
    mesh = plsc.VectorSubcoreMesh(core_axis_name="c", subcore_axis_name="s")
    f = pl.kernel(
        _body,
        out_type=(
            jax.ShapeDtypeStruct((L, 4, 8, 8, 128), jnp.float32),
            jax.ShapeDtypeStruct((B * V, D), jnp.float32),
        ),
        mesh=mesh,
        compiler_params=pltpu.CompilerParams(
            use_tc_tiling_on_sc=False, needs_layout_passes=False),
        scratch_types=[
            pltpu.VMEM((VB, 4, 8, 33), jnp.float32),
            pltpu.VMEM((VB, 4, 8, 33), jnp.float32),
            pltpu.VMEM((VB * 32, 32), jnp.float32),
            pltpu.VMEM((VB * 32, 32), jnp.float32),
            pltpu.VMEM((5, IDX_W), jnp.int32),
            pltpu.VMEM((5, IDX_W), jnp.int32),
            pltpu.VMEM((L, 32), jnp.int32),
            pltpu.VMEM((2 * NCH, IDX_W), jnp.int32),
            pltpu.VMEM((2 * IDX_W, D), jnp.float32),
            pltpu.VMEM((LB, 4, 8, 33), jnp.float32),
            pltpu.SemaphoreType.DMA,
            pltpu.SemaphoreType.DMA,
            pltpu.SemaphoreType.DMA,
            pltpu.SemaphoreType.DMA,
            pltpu.SemaphoreType.DMA,
            pltpu.SemaphoreType.DMA,
        ],
    )
    return f(w5, i2)


def kernel(input, weight):
    w5 = (weight.transpose(1, 2, 0)
          .reshape(V, 4, 8, 8, 128)
          .transpose(0, 1, 3, 2, 4))
    i2 = input.transpose(1, 0).astype(jnp.int32)
    o5, _ = _run(w5, i2)
    return (o5.transpose(0, 1, 3, 2, 4)
            .reshape(L, D, B)
            .transpose(2, 0, 1))


# speedup vs baseline: 1.0723x; 1.0723x over previous
import jax
import jax.numpy as jnp
from jax import lax
from jax.experimental import pallas as pl
from jax.experimental.pallas import tpu as pltpu
from jax.experimental.pallas import tpu_sc as plsc

B, V, D, L = 1024, 1000, 32, 200
NW = 32
NBW = B // NW
VB = 20
NB = V // VB
NPAIR = NB // 2
LB = 4
NCH = L // LB
IDX_W = 128


def _body(w5, i2, o5, tsc, xb0, xb1, yb0, yb1, wx0, wx1, ib, gid, gb, yo,
          semg, semx0, semx1, semw0, semw1, semo):
    c = lax.axis_index("c")
    s = lax.axis_index("s")
    wid = s * 2 + c
    bg = wid >> 2
    bc0 = (wid & 3) * 32
    bfirst = bg * 128 + bc0
    lanes = lax.iota(jnp.int32, 16)
    zero = lanes * 0

    i_dg0 = lanes >> 3
    i_dr = lanes & 7
    bl0 = (bfirst + lanes) * V
    bl1 = (bfirst + 16 + lanes) * V

    def load_slab(v0, xb, semx):
        return pltpu.make_async_copy(
            w5.at[pl.ds(v0, VB), :, bg, :, pl.ds(bc0, 32)],
            xb.at[:, :, :, pl.ds(0, 32)], semx)

    def transpose_batch(v0, xb, yb, wx):
        @plsc.parallel_loop(0, VB, unroll=2)
        def pv(vp):
            xv = xb.at[vp]
            for b4 in range(0, 32, 4):
                gs = []
                for b in range(b4, b4 + 4):
                    ibv = zero + b
                    for h in range(2):
                        gs.append(plsc.load_gather(
                            xv, [i_dg0 + 2 * h, i_dr, ibv]))
                k = 0
                for b in range(b4, b4 + 4):
                    for h in range(2):
                        yb[vp * 32 + b, pl.ds(h * 16, 16)] = gs[k]
                        k += 1
            k0 = vp * 32
            wx[k0 >> 7, pl.ds(k0 & 127, 16)] = bl0 + (v0 + vp)
            k1 = k0 + 16
            wx[k1 >> 7, pl.ds(k1 & 127, 16)] = bl1 + (v0 + vp)

    def scatter_ops(yb, wx, semw):
        return [pltpu.make_async_copy(
            yb.at[pl.ds(p * 128, 128)], tsc.at[wx.at[p]], semw)
            for p in range(5)]

    load_slab(0, xb0, semx0).start()

    def pair(j, carry):
        v0a = j * (2 * VB)
        load_slab(v0a + VB, xb1, semx1).start()
        load_slab(v0a, xb0, semx0).wait()

        @pl.when(j > 0)
        def _():
            for op in scatter_ops(yb0, wx0, semw0):
                op.wait()

        transpose_batch(v0a, xb0, yb0, wx0)
        for op in scatter_ops(yb0, wx0, semw0):
            op.start()

        @pl.when(j + 1 < NPAIR)
        def _():
            load_slab(v0a + 2 * VB, xb0, semx0).start()

        load_slab(v0a + VB, xb1, semx1).wait()

        @pl.when(j > 0)
        def _():
            for op in scatter_ops(yb1, wx1, semw1):
                op.wait()

        transpose_batch(v0a + VB, xb1, yb1, wx1)
        for op in scatter_ops(yb1, wx1, semw1):
            op.start()
        return carry

    lax.fori_loop(0, NPAIR, pair, 0)
    for op in scatter_ops(yb0, wx0, semw0):
        op.wait()
    for op in scatter_ops(yb1, wx1, semw1):
        op.wait()

    pltpu.sync_copy(i2.at[:, pl.ds(bfirst, 32)], ib)

    @plsc.parallel_loop(0, L, unroll=2)
    def p2(li):
        for h in range(2):
            base = (bfirst + h * 16 + lanes) * V
            vv = ib[li, pl.ds(h * 16, 16)]
            pos = li * 32 + h * 16
            gid[pos >> 7, pl.ds(pos & 127, 16)] = vv + base

    def fire_gather(ch, half):
        pltpu.make_async_copy(
            tsc.at[gid.at[ch]], gb.at[pl.ds(half * 128, 128)], semg).start()

    def wait_gather(half):
        pltpu.make_async_copy(
            tsc.at[gid.at[0]], gb.at[pl.ds(half * 128, 128)], semg).wait()

    def wb_op(ch):
        return pltpu.make_async_copy(
            yo.at[:, :, :, pl.ds(0, 32)],
            o5.at[pl.ds(ch * LB, LB), :, bg, :, pl.ds(bc0, 32)], semo)

    fire_gather(0, 0)

    def p23(ch, carry):
        half = ch & 1

        @pl.when(ch + 1 < NCH)
        def _():
            fire_gather(ch + 1, 1 - half)

        wait_gather(half)

        @pl.when(ch >= 1)
        def _():
            wb_op(ch - 1).wait()

        @plsc.parallel_loop(0, LB, unroll=2)
        def p3(lp):
            ilp = zero + lp
            for b4 in range(0, 32, 4):
                gs = []
                for b in range(b4, b4 + 4):
                    for h in range(2):
                        gs.append(gb[half * 128 + lp * 32 + b,
                                     pl.ds(h * 16, 16)])
                k = 0
                for b in range(b4, b4 + 4):
                    ibv = zero + b
                    for h in range(2):
                        plsc.store_scatter(
                            yo, [ilp, i_dg0 + 2 * h, i_dr, ibv], gs[k])
                        k += 1

        wb_op(ch).start()
        return carry

    lax.fori_loop(0, NCH, p23, 0)
    wb_op(NCH - 1).wait()


@jax.jit
def _run(w5, i2):
    mesh = plsc.VectorSubcoreMesh(core_axis_name="c", subcore_axis_name="s")
    f = pl.kernel(
        _body,
        out_type=(
            jax.ShapeDtypeStruct((L, 4, 8, 8, 128), jnp.float32),
            jax.ShapeDtypeStruct((B * V, D), jnp.float32),
        ),
        mesh=mesh,
        compiler_params=pltpu.CompilerParams(
            use_tc_tiling_on_sc=False, needs_layout_passes=False),
        scratch_types=[
            pltpu.VMEM((VB, 4, 8, 33), jnp.float32),
            pltpu.VMEM((VB, 4, 8, 33), jnp.float32),
            pltpu.VMEM((VB * 32, 32), jnp.float32),
            pltpu.VMEM((VB * 32, 32), jnp.float32),
            pltpu.VMEM((5, IDX_W), jnp.int32),
            pltpu.VMEM((5, IDX_W), jnp.int32),
            pltpu.VMEM((L, 32), jnp.int32),
            pltpu.VMEM((2 * NCH, IDX_W), jnp.int32),
            pltpu.VMEM((2 * IDX_W, D), jnp.float32),
            pltpu.VMEM((LB, 4, 8, 33), jnp.float32),
            pltpu.SemaphoreType.DMA,
            pltpu.SemaphoreType.DMA,
            pltpu.SemaphoreType.DMA,
            pltpu.SemaphoreType.DMA,
            pltpu.SemaphoreType.DMA,
            pltpu.SemaphoreType.DMA,
        ],
    )
    return f(w5, i2)


def kernel(input, weight):
    w5 = (weight.transpose(1, 2, 0)
          .reshape(V, 4, 8, 8, 128)
          .transpose(0, 1, 3, 2, 4))
    i2 = input.transpose(1, 0).astype(jnp.int32)
    o5, _ = _run(w5, i2)
    return (o5.transpose(0, 1, 3, 2, 4)
            .reshape(L, D, B)
            .transpose(2, 0, 1))
